# Initial kernel scaffold; baseline (speedup 1.0000x reference)
import functools

import jax
import jax.numpy as jnp
from jax import lax
from jax.experimental import pallas as pl
from jax.experimental.pallas import tpu as pltpu
from jax.experimental.pallas import tpu_sc as plsc


_L = 16


@functools.lru_cache(maxsize=None)
def _make_sc_mean(B: int, S: int, N: int, D: int):
    info = plsc.get_sparse_core_info()
    NC, NS = info.num_cores, info.num_subcores
    NW = NC * NS

    NB = 128 // S
    assert 128 % S == 0 and D % _L == 0
    b_per_w = -(-B // NW)
    b_per_w = -(-b_per_w // NB) * NB
    assert (b_per_w * S) % 8 == 0
    max_blocks = b_per_w // NB
    nd = D // _L
    scale = 1.0 / float(S)

    mesh = plsc.VectorSubcoreMesh(core_axis_name="c", subcore_axis_name="s")

    @functools.partial(
        pl.kernel,
        out_type=jax.ShapeDtypeStruct((B, D), jnp.float32),
        mesh=mesh,
        scratch_types=[
            pltpu.VMEM((NB * S,), jnp.int32),
            pltpu.VMEM((NB * S,), jnp.int32),
            pltpu.VMEM((NB * S, D), jnp.float32),
            pltpu.VMEM((NB * S, D), jnp.float32),
            pltpu.VMEM((NB, D), jnp.float32),
            pltpu.SemaphoreType.DMA,
            pltpu.SemaphoreType.DMA,
        ],
    )
    def sc_mean(idx_hbm, feat_hbm, out_hbm, idx0, idx1, rows0, rows1,
                out_v, sem0, sem1):
        wid = lax.axis_index("s") * NC + lax.axis_index("c")
        base_row = wid * b_per_w
        nblocks = jnp.minimum(max_blocks, (B - base_row) // NB)

        def start_gather(block, idx_v, rows_v, sem):
            off = (base_row + block * NB) * S
            pltpu.sync_copy(idx_hbm.at[pl.ds(off, NB * S)], idx_v)
            pltpu.make_async_copy(feat_hbm.at[idx_v], rows_v, sem).start()

        def wait_gather(idx_v, rows_v, sem):
            pltpu.make_async_copy(feat_hbm.at[idx_v], rows_v, sem).wait()

        def reduce_block(block, rows_v):
            @pl.loop(0, NB)
            def _row(r):
                rbase = r * S
                for d in range(nd):
                    acc = rows_v[rbase, pl.ds(d * _L, _L)]
                    for s in range(1, S):
                        acc = acc + rows_v[rbase + s, pl.ds(d * _L, _L)]
                    out_v[r, pl.ds(d * _L, _L)] = acc * scale
            pltpu.sync_copy(out_v, out_hbm.at[pl.ds(base_row + block * NB, NB)])

        start_gather(0, idx0, rows0, sem0)

        @pl.loop(0, nblocks, step=2)
        def _blocks(i):
            start_gather(i + 1, idx1, rows1, sem1)
            wait_gather(idx0, rows0, sem0)
            reduce_block(i, rows0)
            start_gather(jnp.minimum(i + 2, nblocks - 1), idx0, rows0, sem0)
            wait_gather(idx1, rows1, sem1)
            reduce_block(i + 1, rows1)

        wait_gather(idx0, rows0, sem0)

    return sc_mean


def kernel(nodes, to_neighs, features, num_sample):
    B, S = to_neighs.shape
    N, D = features.shape
    sc_mean = _make_sc_mean(B, S, N, D)
    return sc_mean(to_neighs.reshape(-1).astype(jnp.int32), features)


# speedup vs baseline: 3.8741x; 3.8741x over previous
import functools

import jax
import jax.numpy as jnp
from jax import lax
from jax.experimental import pallas as pl
from jax.experimental.pallas import tpu as pltpu
from jax.experimental.pallas import tpu_sc as plsc


_L = 16


@functools.lru_cache(maxsize=None)
def _make_sc_mean(B: int, S: int, N: int, D: int):
    info = plsc.get_sparse_core_info()
    NC, NS = info.num_cores, info.num_subcores
    NW = NC * NS

    NB = 128 // S
    assert 128 % S == 0 and D % _L == 0
    b_per_w = -(-B // NW)
    b_per_w = -(-b_per_w // (2 * NB)) * (2 * NB)
    assert (b_per_w * S) % 8 == 0
    max_blocks = b_per_w // NB
    for w in range(NW):
        cnt = min(max_blocks, (B - w * b_per_w) // NB)
        assert cnt >= 2 and cnt % 2 == 0, (w, cnt)
    nd = D // _L
    scale = 1.0 / float(S)

    mesh = plsc.VectorSubcoreMesh(core_axis_name="c", subcore_axis_name="s")

    @functools.partial(
        pl.kernel,
        out_type=jax.ShapeDtypeStruct((B, D), jnp.float32),
        mesh=mesh,
        scratch_types=[
            pltpu.VMEM((NB * S,), jnp.int32),
            pltpu.VMEM((NB * S,), jnp.int32),
            pltpu.VMEM((NB * S, D), jnp.float32),
            pltpu.VMEM((NB * S, D), jnp.float32),
            pltpu.VMEM((NB, D), jnp.float32),
            pltpu.SemaphoreType.DMA,
            pltpu.SemaphoreType.DMA,
        ],
    )
    def sc_mean(idx_hbm, feat_hbm, out_hbm, idx0, idx1, rows0, rows1,
                out_v, sem0, sem1):
        wid = lax.axis_index("s") * NC + lax.axis_index("c")
        base_row = wid * b_per_w
        nblocks = jnp.minimum(max_blocks, (B - base_row) // NB)

        def start_gather(block, idx_v, rows_v, sem):
            off = (base_row + block * NB) * S
            pltpu.sync_copy(idx_hbm.at[pl.ds(off, NB * S)], idx_v)
            pltpu.make_async_copy(feat_hbm.at[idx_v], rows_v, sem).start()

        def wait_gather(idx_v, rows_v, sem):
            pltpu.make_async_copy(feat_hbm.at[idx_v], rows_v, sem).wait()

        def reduce_block(block, rows_v):
            @pl.loop(0, NB)
            def _row(r):
                rbase = r * S
                for d in range(nd):
                    acc = rows_v[rbase, pl.ds(d * _L, _L)]
                    for s in range(1, S):
                        acc = acc + rows_v[rbase + s, pl.ds(d * _L, _L)]
                    out_v[r, pl.ds(d * _L, _L)] = acc * scale
            pltpu.sync_copy(out_v, out_hbm.at[pl.ds(base_row + block * NB, NB)])

        start_gather(0, idx0, rows0, sem0)

        @pl.loop(0, nblocks, step=2)
        def _blocks(i):
            start_gather(i + 1, idx1, rows1, sem1)
            wait_gather(idx0, rows0, sem0)
            reduce_block(i, rows0)
            start_gather(jnp.minimum(i + 2, nblocks - 1), idx0, rows0, sem0)
            wait_gather(idx1, rows1, sem1)
            reduce_block(i + 1, rows1)

        wait_gather(idx0, rows0, sem0)

    return sc_mean


def kernel(nodes, to_neighs, features, num_sample):
    B, S = to_neighs.shape
    N, D = features.shape
    sc_mean = _make_sc_mean(B, S, N, D)
    return sc_mean(to_neighs.reshape(-1).astype(jnp.int32), features)


# speedup vs baseline: 5.3181x; 1.3727x over previous
import functools

import jax
import jax.numpy as jnp
from jax import lax
from jax.experimental import pallas as pl
from jax.experimental.pallas import tpu as pltpu
from jax.experimental.pallas import tpu_sc as plsc


_L = 16


@functools.lru_cache(maxsize=None)
def _make_sc_mean(B: int, S: int, N: int, D: int):
    info = plsc.get_sparse_core_info()
    NC, NS = info.num_cores, info.num_subcores
    NW = NC * NS

    NB = 128 // S
    assert 128 % S == 0 and D % _L == 0
    b_per_w = -(-B // NW)
    b_per_w = -(-b_per_w // (2 * NB)) * (2 * NB)
    max_blocks = b_per_w // NB
    tail_rows = B - (B // b_per_w) * b_per_w if B % b_per_w else b_per_w
    chunk_rows = 1
    for c in range(min(tail_rows, b_per_w), 0, -1):
        if tail_rows % c == 0 and b_per_w % c == 0:
            chunk_rows = c
            break
    n_chunks = b_per_w // chunk_rows
    for w in range(NW):
        cnt = min(max_blocks, (B - w * b_per_w) // NB)
        assert cnt >= 2 and cnt % 2 == 0, (w, cnt)
    nd = D // _L
    scale = 1.0 / float(S)

    mesh = plsc.VectorSubcoreMesh(core_axis_name="c", subcore_axis_name="s")

    @functools.partial(
        pl.kernel,
        out_type=jax.ShapeDtypeStruct((B, D), jnp.float32),
        mesh=mesh,
        scratch_types=[
            pltpu.VMEM((max_blocks, NB * S), jnp.int32),
            pltpu.VMEM((NB * S, D), jnp.float32),
            pltpu.VMEM((NB * S, D), jnp.float32),
            pltpu.VMEM((b_per_w, D), jnp.float32),
            pltpu.SemaphoreType.DMA,
            pltpu.SemaphoreType.DMA,
        ],
    )
    def sc_mean(idx_hbm, feat_hbm, out_hbm, idx_v, rows0, rows1,
                out_v, sem0, sem1):
        wid = lax.axis_index("s") * NC + lax.axis_index("c")
        base_row = wid * b_per_w
        nblocks = jnp.minimum(max_blocks, (B - base_row) // NB)

        pltpu.sync_copy(idx_hbm.at[wid], idx_v)

        def start_gather(block, rows_v, sem):
            pltpu.make_async_copy(
                feat_hbm.at[idx_v.at[block]], rows_v, sem).start()

        def wait_gather(block, rows_v, sem):
            pltpu.make_async_copy(
                feat_hbm.at[idx_v.at[block]], rows_v, sem).wait()

        def reduce_block(block, rows_v):
            @pl.loop(0, NB)
            def _row(r):
                rbase = r * S
                accs = [rows_v[rbase, pl.ds(d * _L, _L)] for d in range(nd)]
                for s in range(1, S):
                    accs = [accs[d] + rows_v[rbase + s, pl.ds(d * _L, _L)]
                            for d in range(nd)]
                orow = block * NB + r
                for d in range(nd):
                    out_v[orow, pl.ds(d * _L, _L)] = accs[d] * scale

        start_gather(0, rows0, sem0)

        @pl.loop(0, nblocks, step=2)
        def _blocks(i):
            start_gather(i + 1, rows1, sem1)
            wait_gather(i, rows0, sem0)
            reduce_block(i, rows0)
            start_gather(jnp.minimum(i + 2, nblocks - 1), rows0, sem0)
            wait_gather(i + 1, rows1, sem1)
            reduce_block(i + 1, rows1)

        wait_gather(0, rows0, sem0)

        valid_rows = nblocks * NB
        for c in range(n_chunks):
            @pl.when((c + 1) * chunk_rows <= valid_rows)
            def _flush():
                pltpu.sync_copy(
                    out_v.at[pl.ds(c * chunk_rows, chunk_rows)],
                    out_hbm.at[pl.ds(base_row + c * chunk_rows, chunk_rows)])

    def call(to_neighs, features):
        idx = to_neighs.reshape(-1).astype(jnp.int32)
        total = NW * max_blocks * NB * S
        if total > idx.size:
            idx = jnp.concatenate(
                [idx, jnp.zeros((total - idx.size,), jnp.int32)])
        return sc_mean(idx.reshape(NW, max_blocks, NB * S), features)

    return call


def kernel(nodes, to_neighs, features, num_sample):
    B, S = to_neighs.shape
    N, D = features.shape
    return _make_sc_mean(B, S, N, D)(to_neighs, features)


# speedup vs baseline: 5.8668x; 1.1032x over previous
import functools

import jax
import jax.numpy as jnp
from jax import lax
from jax.experimental import pallas as pl
from jax.experimental.pallas import tpu as pltpu
from jax.experimental.pallas import tpu_sc as plsc


_L = 16


@functools.lru_cache(maxsize=None)
def _make_sc_mean(B: int, S: int, N: int, D: int):
    info = plsc.get_sparse_core_info()
    NC, NS = info.num_cores, info.num_subcores
    NW = NC * NS

    NB = 256 // S
    SPB = NB * S // 128
    assert 128 % S == 0 and D % _L == 0
    b_per_w = -(-B // NW)
    b_per_w = -(-b_per_w // (2 * NB)) * (2 * NB)
    max_blocks = b_per_w // NB
    tail_rows = B - (B // b_per_w) * b_per_w if B % b_per_w else b_per_w
    chunk_rows = 1
    for c in range(min(tail_rows, b_per_w), 0, -1):
        if tail_rows % c == 0 and b_per_w % c == 0:
            chunk_rows = c
            break
    n_chunks = b_per_w // chunk_rows
    for w in range(NW):
        cnt = min(max_blocks, (B - w * b_per_w) // NB)
        assert cnt >= 2 and cnt % 2 == 0, (w, cnt)
    nd = D // _L
    scale = 1.0 / float(S)

    mesh = plsc.VectorSubcoreMesh(core_axis_name="c", subcore_axis_name="s")

    @functools.partial(
        pl.kernel,
        out_type=jax.ShapeDtypeStruct((B, D), jnp.float32),
        mesh=mesh,
        scratch_types=[
            pltpu.VMEM((max_blocks * SPB, 128), jnp.int32),
            pltpu.VMEM((NB * S, D), jnp.float32),
            pltpu.VMEM((NB * S, D), jnp.float32),
            pltpu.VMEM((b_per_w, D), jnp.float32),
            pltpu.SemaphoreType.DMA,
            pltpu.SemaphoreType.DMA,
        ],
    )
    def sc_mean(idx_hbm, feat_hbm, out_hbm, idx_v, rows0, rows1,
                out_v, sem0, sem1):
        wid = lax.axis_index("s") * NC + lax.axis_index("c")
        base_row = wid * b_per_w
        nblocks = jnp.minimum(max_blocks, (B - base_row) // NB)

        pltpu.sync_copy(idx_hbm.at[wid], idx_v)

        def start_gather(block, rows_v, sem):
            for j in range(SPB):
                pltpu.make_async_copy(
                    feat_hbm.at[idx_v.at[block * SPB + j]],
                    rows_v.at[pl.ds(j * 128, 128)], sem).start()

        def wait_gather(block, rows_v, sem):
            for j in range(SPB):
                pltpu.make_async_copy(
                    feat_hbm.at[idx_v.at[block * SPB + j]],
                    rows_v.at[pl.ds(j * 128, 128)], sem).wait()

        def reduce_block(block, rows_v):
            @pl.loop(0, NB)
            def _row(r):
                rbase = r * S
                accs = [rows_v[rbase, pl.ds(d * _L, _L)] for d in range(nd)]
                for s in range(1, S):
                    accs = [accs[d] + rows_v[rbase + s, pl.ds(d * _L, _L)]
                            for d in range(nd)]
                orow = block * NB + r
                for d in range(nd):
                    out_v[orow, pl.ds(d * _L, _L)] = accs[d] * scale

        start_gather(0, rows0, sem0)

        @pl.loop(0, nblocks, step=2)
        def _blocks(i):
            start_gather(i + 1, rows1, sem1)
            wait_gather(i, rows0, sem0)
            reduce_block(i, rows0)
            start_gather(jnp.minimum(i + 2, nblocks - 1), rows0, sem0)
            wait_gather(i + 1, rows1, sem1)
            reduce_block(i + 1, rows1)

        wait_gather(0, rows0, sem0)

        valid_rows = nblocks * NB
        for c in range(n_chunks):
            @pl.when((c + 1) * chunk_rows <= valid_rows)
            def _flush():
                pltpu.sync_copy(
                    out_v.at[pl.ds(c * chunk_rows, chunk_rows)],
                    out_hbm.at[pl.ds(base_row + c * chunk_rows, chunk_rows)])

    def call(to_neighs, features):
        idx = to_neighs.reshape(-1).astype(jnp.int32)
        total = NW * max_blocks * NB * S
        if total > idx.size:
            idx = jnp.concatenate(
                [idx, jnp.zeros((total - idx.size,), jnp.int32)])
        return sc_mean(idx.reshape(NW, max_blocks * SPB, 128), features)

    return call


def kernel(nodes, to_neighs, features, num_sample):
    B, S = to_neighs.shape
    N, D = features.shape
    return _make_sc_mean(B, S, N, D)(to_neighs, features)
